# Initial kernel scaffold; baseline (speedup 1.0000x reference)
import jax
import jax.numpy as jnp
from jax.experimental import pallas as pl


def kernel(x, w_t, b2):
    raise NotImplementedError("write your pallas kernel here")


# speedup vs baseline: 5.4169x; 5.4169x over previous
import functools

import jax
import jax.numpy as jnp
from jax.experimental import pallas as pl
from jax.experimental.pallas import tpu as pltpu


def _linear_bf16_kernel(x_ref, w_ref, b_ref, o_ref):
    o_ref[...] = (
        jnp.dot(x_ref[...], w_ref[...], preferred_element_type=jnp.float32)
        + b_ref[...]
    ).astype(o_ref.dtype)


@functools.partial(jax.jit, static_argnames=("tm", "tn"))
def _linear_call(x_bf, w_bf, b2, tm, tn):
    M, K = x_bf.shape
    _, N = w_bf.shape
    grid = (pl.cdiv(M, tm), pl.cdiv(N, tn))
    return pl.pallas_call(
        _linear_bf16_kernel,
        out_shape=jax.ShapeDtypeStruct((M, N), jnp.float32),
        grid_spec=pltpu.PrefetchScalarGridSpec(
            num_scalar_prefetch=0,
            grid=grid,
            in_specs=[
                pl.BlockSpec((tm, K), lambda i, j: (i, 0)),
                pl.BlockSpec((K, tn), lambda i, j: (0, j)),
                pl.BlockSpec((1, tn), lambda i, j: (0, j)),
            ],
            out_specs=pl.BlockSpec((tm, tn), lambda i, j: (i, j)),
        ),
        compiler_params=pltpu.CompilerParams(
            dimension_semantics=("parallel", "parallel"),
            vmem_limit_bytes=60 * 1024 * 1024,
        ),
    )(x_bf, w_bf, b2)


def kernel(x, w_t, b2):
    x_bf = x.astype(jnp.bfloat16)
    w_bf = w_t.astype(jnp.bfloat16)
    return _linear_call(x_bf, w_bf, b2, tm=1024, tn=1024)


# speedup vs baseline: 6.3256x; 1.1677x over previous
import functools

import jax
import jax.numpy as jnp
from jax.experimental import pallas as pl
from jax.experimental.pallas import tpu as pltpu


def _linear_kernel(x_ref, w_ref, b_ref, o_ref):
    x = x_ref[...].astype(jnp.bfloat16)
    o_ref[...] = (
        jnp.dot(x, w_ref[...], preferred_element_type=jnp.float32)
        + b_ref[...]
    ).astype(o_ref.dtype)


@functools.partial(jax.jit, static_argnames=("tm",))
def _linear_call(x, w_bf, b2, tm):
    M, K = x.shape
    _, N = w_bf.shape
    grid = (pl.cdiv(M, tm),)
    return pl.pallas_call(
        _linear_kernel,
        out_shape=jax.ShapeDtypeStruct((M, N), jnp.float32),
        grid_spec=pltpu.PrefetchScalarGridSpec(
            num_scalar_prefetch=0,
            grid=grid,
            in_specs=[
                pl.BlockSpec((tm, K), lambda i: (i, 0)),
                pl.BlockSpec((K, N), lambda i: (0, 0)),
                pl.BlockSpec((1, N), lambda i: (0, 0)),
            ],
            out_specs=pl.BlockSpec((tm, N), lambda i: (i, 0)),
        ),
        compiler_params=pltpu.CompilerParams(
            dimension_semantics=("parallel",),
            vmem_limit_bytes=60 * 1024 * 1024,
        ),
    )(x, w_bf, b2)


def kernel(x, w_t, b2):
    w_bf = w_t.astype(jnp.bfloat16)
    return _linear_call(x, w_bf, b2, tm=256)
